# Initial kernel scaffold; baseline (speedup 1.0000x reference)
import functools

import jax
import jax.numpy as jnp
from jax import lax
from jax.experimental import pallas as pl
from jax.experimental.pallas import tpu as pltpu
from jax.experimental.pallas import tpu_sc as plsc

N = 10000
E = 320000
C1 = 32
TW = 48
NC, NS, L = 2, 16, 16
NW = NC * NS
EPW = E // NW
K = 80
NCHUNK = EPW // K
RPT = N // NS

_MESH = plsc.VectorSubcoreMesh(core_axis_name="c", subcore_axis_name="s",
                               num_cores=NC, num_subcores=NS)


def _sc_gat_body(src_hbm, dst_hbm, xl_hbm, xr_hbm, att_hbm, zer_hbm, out_hbm,
                 idx_s, idx_d, gbs, gbd, srow, attv, table, sem):
    cid = lax.axis_index("c")
    sid = lax.axis_index("s")
    wid = sid * NC + cid
    pltpu.sync_copy(zer_hbm.at[pl.ds(sid * RPT, RPT)],
                    table.at[pl.ds(sid * RPT, RPT)])
    pltpu.sync_copy(att_hbm, attv)
    plsc.subcore_barrier()

    zero16 = jnp.zeros((L,), jnp.float32)
    oneh = jnp.where(lax.iota(jnp.int32, L) == 0, 1.0, 0.0)
    base = wid * EPW

    def chunk(c, carry):
        r0 = base + c * K
        pltpu.sync_copy(src_hbm.at[pl.ds(r0, K)], idx_s)
        pltpu.sync_copy(dst_hbm.at[pl.ds(r0, K)], idx_d)
        pltpu.async_copy(xl_hbm.at[idx_s], gbs, sem).wait()
        pltpu.async_copy(xr_hbm.at[idx_d], gbd, sem).wait()
        a0 = attv[pl.ds(0, L)]
        a1 = attv[pl.ds(L, L)]

        def edge(i, carry2):
            s0 = gbs[i, pl.ds(0, L)]
            s1 = gbs[i, pl.ds(L, L)]
            z0 = s0 + gbd[i, pl.ds(0, L)]
            z1 = s1 + gbd[i, pl.ds(L, L)]
            z0 = jnp.maximum(z0, 0.0) + 0.2 * jnp.minimum(z0, 0.0)
            z1 = jnp.maximum(z1, 0.0) + 0.2 * jnp.minimum(z1, 0.0)
            tot = jnp.sum(z0 * a0 + z1 * a1)
            gv = jnp.exp(tot + zero16)
            srow[i, pl.ds(0, L)] = gv * s0
            srow[i, pl.ds(L, L)] = gv * s1
            srow[i, pl.ds(2 * L, L)] = gv * oneh
            return carry2

        lax.fori_loop(0, K, edge, 0)
        pltpu.sync_copy(srow, table.at[idx_d], add=True)
        return carry

    lax.fori_loop(0, NCHUNK, chunk, 0)
    plsc.subcore_barrier()
    pltpu.sync_copy(table.at[pl.ds(sid * RPT, RPT)],
                    out_hbm.at[cid, pl.ds(sid * RPT, RPT)])


_sc_gat = pl.kernel(
    _sc_gat_body,
    out_type=jax.ShapeDtypeStruct((NC, N, TW), jnp.float32),
    mesh=_MESH,
    scratch_types=[
        pltpu.VMEM((K,), jnp.int32),
        pltpu.VMEM((K,), jnp.int32),
        pltpu.VMEM((K, C1), jnp.float32),
        pltpu.VMEM((K, C1), jnp.float32),
        pltpu.VMEM((K, TW), jnp.float32),
        pltpu.VMEM((C1,), jnp.float32),
        pltpu.VMEM_SHARED((N, TW), jnp.float32),
        pltpu.SemaphoreType.DMA,
    ],
)


def _sc_agg_body(src_hbm, dst_hbm, h1_hbm, zer_hbm, o1_hbm, o2_hbm,
                 idx_s, idx_d, bufs, bufd, t1, t2, sem):
    cid = lax.axis_index("c")
    sid = lax.axis_index("s")
    wid = sid * NC + cid
    pltpu.sync_copy(zer_hbm.at[pl.ds(sid * RPT, RPT)],
                    t1.at[pl.ds(sid * RPT, RPT)])
    pltpu.sync_copy(zer_hbm.at[pl.ds(sid * RPT, RPT)],
                    t2.at[pl.ds(sid * RPT, RPT)])
    plsc.subcore_barrier()
    base = wid * EPW

    def chunk(c, carry):
        r0 = base + c * K
        pltpu.sync_copy(src_hbm.at[pl.ds(r0, K)], idx_s)
        pltpu.sync_copy(dst_hbm.at[pl.ds(r0, K)], idx_d)
        pltpu.async_copy(h1_hbm.at[idx_d], bufd, sem).wait()
        pltpu.async_copy(h1_hbm.at[idx_s], bufs, sem).wait()
        pltpu.sync_copy(bufd, t1.at[idx_s], add=True)
        pltpu.sync_copy(bufs, t2.at[idx_d], add=True)
        return carry

    lax.fori_loop(0, NCHUNK, chunk, 0)
    plsc.subcore_barrier()
    pltpu.sync_copy(t1.at[pl.ds(sid * RPT, RPT)],
                    o1_hbm.at[cid, pl.ds(sid * RPT, RPT)])
    pltpu.sync_copy(t2.at[pl.ds(sid * RPT, RPT)],
                    o2_hbm.at[cid, pl.ds(sid * RPT, RPT)])


_sc_agg = pl.kernel(
    _sc_agg_body,
    out_type=[jax.ShapeDtypeStruct((NC, N, TW), jnp.float32),
              jax.ShapeDtypeStruct((NC, N, TW), jnp.float32)],
    mesh=_MESH,
    scratch_types=[
        pltpu.VMEM((K,), jnp.int32),
        pltpu.VMEM((K,), jnp.int32),
        pltpu.VMEM((K, TW), jnp.float32),
        pltpu.VMEM((K, TW), jnp.float32),
        pltpu.VMEM_SHARED((N, TW), jnp.float32),
        pltpu.VMEM_SHARED((N, TW), jnp.float32),
        pltpu.SemaphoreType.DMA,
    ],
)


def _sc_fin_body(src_hbm, dst_hbm, t1_hbm, t2_hbm, out_hbm,
                 idx_s, idx_d, bufa, bufb, obuf, sem):
    cid = lax.axis_index("c")
    sid = lax.axis_index("s")
    wid = sid * NC + cid
    base = wid * EPW

    def chunk(c, carry):
        r0 = base + c * K
        pltpu.sync_copy(src_hbm.at[pl.ds(r0, K)], idx_s)
        pltpu.sync_copy(dst_hbm.at[pl.ds(r0, K)], idx_d)
        pltpu.async_copy(t1_hbm.at[idx_s], bufa, sem).wait()
        pltpu.async_copy(t2_hbm.at[idx_d], bufb, sem).wait()

        def edge(i, carry2):
            for j in range(4):
                v = bufa[i, pl.ds(j * L, L)] + bufb[i, pl.ds(j * L, L)]
                obuf[i, pl.ds(j * L, L)] = jnp.maximum(v, 0.0)
            for j in range(4, 8):
                obuf[i, pl.ds(j * L, L)] = (bufa[i, pl.ds(j * L, L)]
                                            + bufb[i, pl.ds(j * L, L)])
            return carry2

        lax.fori_loop(0, K, edge, 0)
        pltpu.sync_copy(obuf, out_hbm.at[pl.ds(r0, K)])
        return carry

    lax.fori_loop(0, NCHUNK, chunk, 0)


_sc_fin = pl.kernel(
    _sc_fin_body,
    out_type=jax.ShapeDtypeStruct((E, 128), jnp.float32),
    mesh=_MESH,
    scratch_types=[
        pltpu.VMEM((K,), jnp.int32),
        pltpu.VMEM((K,), jnp.int32),
        pltpu.VMEM((K, 128), jnp.float32),
        pltpu.VMEM((K, 128), jnp.float32),
        pltpu.VMEM((K, 128), jnp.float32),
        pltpu.SemaphoreType.DMA,
    ],
)


def _tc1_body(x_ref, wl_ref, bl_ref, wr_ref, br_ref, att_ref,
              xl_ref, xr_ref, selfr_ref):
    x = x_ref[...]
    xl = jnp.dot(x, wl_ref[...], preferred_element_type=jnp.float32) + bl_ref[...]
    xr = jnp.dot(x, wr_ref[...], preferred_element_type=jnp.float32) + br_ref[...]
    xl_ref[...] = xl
    xr_ref[...] = xr
    z = xl + xr
    z = jnp.maximum(z, 0.0) + 0.2 * jnp.minimum(z, 0.0)
    g = jnp.exp(jnp.dot(z, att_ref[...], preferred_element_type=jnp.float32))
    selfr_ref[...] = jnp.concatenate(
        [g * xl, g, jnp.zeros((N, TW - C1 - 1), jnp.float32)], axis=1)


def _tc2_body(pa_ref, selfr_ref, h1aug_ref):
    t = pa_ref[0] + pa_ref[1] + selfr_ref[...]
    num = t[:, :C1]
    den = t[:, C1:C1 + 1]
    gat = num / den
    gat = gat - jnp.mean(gat, axis=0, keepdims=True)
    h1 = jnp.maximum(
        gat / jnp.sqrt(1e-5 + jnp.mean(jnp.sum(gat * gat, axis=-1))), 0.0)
    h1aug_ref[...] = jnp.concatenate(
        [h1, jnp.ones((N, 1), jnp.float32),
         jnp.zeros((N, TW - C1 - 1), jnp.float32)], axis=1)


def _tc3_body(s1_ref, s2_ref, h1aug_ref, w1_ref, b1_ref, fc_ref,
              t1_ref, t2_ref):
    s1 = s1_ref[0] + s1_ref[1]
    s2 = s2_ref[0] + s2_ref[1]
    h1 = h1aug_ref[:, :C1]
    deg_s = s1[:, C1:C1 + 1]
    deg_d = s2[:, C1:C1 + 1]
    agg = jnp.concatenate([deg_s * h1 + s2[:, :C1],
                           s1[:, :C1] + deg_d * h1], axis=1)
    w1 = w1_ref[...]
    wa, wb, wc = w1[:64], w1[64:128], w1[128:192]
    wx = wa - wb - wc
    fc = fc_ref[...]
    cs = (jnp.dot(h1, wx[:C1], preferred_element_type=jnp.float32)
          + jnp.dot(agg, wb, preferred_element_type=jnp.float32) + b1_ref[...])
    cd = (jnp.dot(h1, wx[C1:], preferred_element_type=jnp.float32)
          + jnp.dot(agg, wc, preferred_element_type=jnp.float32))
    ds = jnp.dot(h1, fc[:C1], preferred_element_type=jnp.float32)
    dd = jnp.dot(h1, fc[C1:], preferred_element_type=jnp.float32)
    t1_ref[...] = jnp.concatenate([cs, ds], axis=1)
    t2_ref[...] = jnp.concatenate([cd, dd], axis=1)


_TC4_R = 4000


def _tc4_body(uw_ref, w2_ref, fc_ref, fcb_ref, b2_ref, ow_ref, ob_ref,
              out_ref):
    fc = fc_ref[...]
    m = jnp.dot(w2_ref[...], fc, preferred_element_type=jnp.float32)
    c0 = jnp.dot(b2_ref[...], fc, preferred_element_type=jnp.float32) + fcb_ref[...]
    uw = uw_ref[...]
    u = uw[:, :64]
    w = uw[:, 64:]
    v = jnp.maximum(jnp.dot(u, m, preferred_element_type=jnp.float32) + w + c0,
                    0.0)
    out_ref[...] = (jnp.dot(v, ow_ref[...], preferred_element_type=jnp.float32)
                    + ob_ref[...])


def _full_spec(shape):
    return pl.BlockSpec(shape, lambda i: tuple(0 for _ in shape))


@jax.jit
def kernel(x, edge_index, edge_attr, Wl1, bl1, Wr1, br1, att1, bias1,
           mp_W1, mp_b1, mp_W2, mp_b2, fc_W1, fc_b1, out_W, out_b):
    del edge_attr, bias1
    src = edge_index[0]
    dst = edge_index[1]
    zer = jnp.zeros((N, TW), jnp.float32)

    xl, xr, selfr = pl.pallas_call(
        _tc1_body,
        out_shape=[jax.ShapeDtypeStruct((N, C1), jnp.float32),
                   jax.ShapeDtypeStruct((N, C1), jnp.float32),
                   jax.ShapeDtypeStruct((N, TW), jnp.float32)],
    )(x, Wl1, bl1.reshape(1, C1), Wr1, br1.reshape(1, C1),
      att1.reshape(C1, 1))

    pa = _sc_gat(src, dst, xl, xr, att1, zer)

    h1aug = pl.pallas_call(
        _tc2_body,
        out_shape=jax.ShapeDtypeStruct((N, TW), jnp.float32),
    )(pa, selfr)

    s1, s2 = _sc_agg(src, dst, h1aug, zer)

    t1, t2 = pl.pallas_call(
        _tc3_body,
        out_shape=[jax.ShapeDtypeStruct((N, 128), jnp.float32),
                   jax.ShapeDtypeStruct((N, 128), jnp.float32)],
    )(s1, s2, h1aug, mp_W1, mp_b1.reshape(1, 64), fc_W1)

    uw = _sc_fin(src, dst, t1, t2)

    out = pl.pallas_call(
        _tc4_body,
        grid=(E // _TC4_R,),
        in_specs=[
            pl.BlockSpec((_TC4_R, 128), lambda i: (i, 0)),
            _full_spec((64, 64)),
            _full_spec((64, 64)),
            _full_spec((1, 64)),
            _full_spec((1, 64)),
            _full_spec((64, 16)),
            _full_spec((1, 16)),
        ],
        out_specs=pl.BlockSpec((_TC4_R, 16), lambda i: (i, 0)),
        out_shape=jax.ShapeDtypeStruct((E, 16), jnp.float32),
    )(uw, mp_W2, fc_W1, fc_b1.reshape(1, 64), mp_b2.reshape(1, 64),
      out_W, out_b.reshape(1, 16))

    return out


# speedup vs baseline: 5.3783x; 5.3783x over previous
import functools

import jax
import jax.numpy as jnp
from jax import lax
from jax.experimental import pallas as pl
from jax.experimental.pallas import tpu as pltpu
from jax.experimental.pallas import tpu_sc as plsc

N = 10000
NPAD = 10240
E = 320000
C1 = 32
TW = 48
NC, NS, L = 2, 16, 16
NW = NC * NS
EPW = E // NW
K = 80
NCHUNK = EPW // K
RPT = NPAD // NS

_MESH = plsc.VectorSubcoreMesh(core_axis_name="c", subcore_axis_name="s",
                               num_cores=NC, num_subcores=NS)
_SC_PARAMS = pltpu.CompilerParams(needs_layout_passes=False,
                                  use_tc_tiling_on_sc=False)


def _sc_gat_body(src_hbm, dst_hbm, xl_hbm, xr_hbm, att_hbm, zer_hbm, out_hbm,
                 idx_s, idx_d, gbs, gbd, srow, attv, table, sem):
    cid = lax.axis_index("c")
    sid = lax.axis_index("s")
    wid = sid * NC + cid
    pltpu.sync_copy(zer_hbm.at[pl.ds(sid * RPT, RPT)],
                    table.at[pl.ds(sid * RPT, RPT)])
    pltpu.sync_copy(att_hbm, attv)
    plsc.subcore_barrier()

    zero16 = jnp.zeros((L,), jnp.float32)
    oneh = jnp.where(lax.iota(jnp.int32, L) == 0, 1.0, 0.0)
    base = wid * EPW

    def chunk(c, carry):
        r0 = base + c * K
        pltpu.sync_copy(src_hbm.at[pl.ds(r0, K)], idx_s)
        pltpu.sync_copy(dst_hbm.at[pl.ds(r0, K)], idx_d)
        pltpu.async_copy(xl_hbm.at[idx_s], gbs, sem).wait()
        pltpu.async_copy(xr_hbm.at[idx_d], gbd, sem).wait()
        a0 = attv[pl.ds(0, L)]
        a1 = attv[pl.ds(L, L)]

        def edge(i, carry2):
            s0 = gbs[i, pl.ds(0, L)]
            s1 = gbs[i, pl.ds(L, L)]
            z0 = s0 + gbd[i, pl.ds(0, L)]
            z1 = s1 + gbd[i, pl.ds(L, L)]
            z0 = jnp.maximum(z0, 0.0) + 0.2 * jnp.minimum(z0, 0.0)
            z1 = jnp.maximum(z1, 0.0) + 0.2 * jnp.minimum(z1, 0.0)
            tot = jnp.sum(z0 * a0 + z1 * a1)
            gv = jnp.exp(tot + zero16)
            srow[i, pl.ds(0, L)] = gv * s0
            srow[i, pl.ds(L, L)] = gv * s1
            srow[i, pl.ds(2 * L, L)] = gv * oneh
            return carry2

        lax.fori_loop(0, K, edge, 0)
        pltpu.sync_copy(srow, table.at[idx_d], add=True)
        return carry

    lax.fori_loop(0, NCHUNK, chunk, 0)
    plsc.subcore_barrier()
    pltpu.sync_copy(table.at[pl.ds(sid * RPT, RPT)],
                    out_hbm.at[cid, pl.ds(sid * RPT, RPT)])


_sc_gat = pl.kernel(
    _sc_gat_body,
    out_type=jax.ShapeDtypeStruct((NC, NPAD, TW), jnp.float32),
    mesh=_MESH,
    compiler_params=_SC_PARAMS,
    scratch_types=[
        pltpu.VMEM((K,), jnp.int32),
        pltpu.VMEM((K,), jnp.int32),
        pltpu.VMEM((K, C1), jnp.float32),
        pltpu.VMEM((K, C1), jnp.float32),
        pltpu.VMEM((K, TW), jnp.float32),
        pltpu.VMEM((C1,), jnp.float32),
        pltpu.VMEM_SHARED((NPAD, TW), jnp.float32),
        pltpu.SemaphoreType.DMA,
    ],
)


def _sc_agg_body(src_hbm, dst_hbm, h1_hbm, zer_hbm, o1_hbm, o2_hbm,
                 idx_s, idx_d, bufs, bufd, t1, t2, sem):
    cid = lax.axis_index("c")
    sid = lax.axis_index("s")
    wid = sid * NC + cid
    pltpu.sync_copy(zer_hbm.at[pl.ds(sid * RPT, RPT)],
                    t1.at[pl.ds(sid * RPT, RPT)])
    pltpu.sync_copy(zer_hbm.at[pl.ds(sid * RPT, RPT)],
                    t2.at[pl.ds(sid * RPT, RPT)])
    plsc.subcore_barrier()
    base = wid * EPW

    def chunk(c, carry):
        r0 = base + c * K
        pltpu.sync_copy(src_hbm.at[pl.ds(r0, K)], idx_s)
        pltpu.sync_copy(dst_hbm.at[pl.ds(r0, K)], idx_d)
        pltpu.async_copy(h1_hbm.at[idx_d], bufd, sem).wait()
        pltpu.async_copy(h1_hbm.at[idx_s], bufs, sem).wait()
        pltpu.sync_copy(bufd, t1.at[idx_s], add=True)
        pltpu.sync_copy(bufs, t2.at[idx_d], add=True)
        return carry

    lax.fori_loop(0, NCHUNK, chunk, 0)
    plsc.subcore_barrier()
    pltpu.sync_copy(t1.at[pl.ds(sid * RPT, RPT)],
                    o1_hbm.at[cid, pl.ds(sid * RPT, RPT)])
    pltpu.sync_copy(t2.at[pl.ds(sid * RPT, RPT)],
                    o2_hbm.at[cid, pl.ds(sid * RPT, RPT)])


_sc_agg = pl.kernel(
    _sc_agg_body,
    out_type=[jax.ShapeDtypeStruct((NC, NPAD, TW), jnp.float32),
              jax.ShapeDtypeStruct((NC, NPAD, TW), jnp.float32)],
    mesh=_MESH,
    compiler_params=_SC_PARAMS,
    scratch_types=[
        pltpu.VMEM((K,), jnp.int32),
        pltpu.VMEM((K,), jnp.int32),
        pltpu.VMEM((K, TW), jnp.float32),
        pltpu.VMEM((K, TW), jnp.float32),
        pltpu.VMEM_SHARED((NPAD, TW), jnp.float32),
        pltpu.VMEM_SHARED((NPAD, TW), jnp.float32),
        pltpu.SemaphoreType.DMA,
    ],
)


def _sc_fin_body(src_hbm, dst_hbm, t1_hbm, t2_hbm, out_hbm,
                 idx_s, idx_d, bufa, bufb, obuf, sem):
    cid = lax.axis_index("c")
    sid = lax.axis_index("s")
    wid = sid * NC + cid
    base = wid * EPW

    def chunk(c, carry):
        r0 = base + c * K
        pltpu.sync_copy(src_hbm.at[pl.ds(r0, K)], idx_s)
        pltpu.sync_copy(dst_hbm.at[pl.ds(r0, K)], idx_d)
        pltpu.async_copy(t1_hbm.at[idx_s], bufa, sem).wait()
        pltpu.async_copy(t2_hbm.at[idx_d], bufb, sem).wait()

        def edge(i, carry2):
            for j in range(4):
                v = bufa[i, pl.ds(j * L, L)] + bufb[i, pl.ds(j * L, L)]
                obuf[i, pl.ds(j * L, L)] = jnp.maximum(v, 0.0)
            for j in range(4, 8):
                obuf[i, pl.ds(j * L, L)] = (bufa[i, pl.ds(j * L, L)]
                                            + bufb[i, pl.ds(j * L, L)])
            return carry2

        lax.fori_loop(0, K, edge, 0)
        pltpu.sync_copy(obuf, out_hbm.at[pl.ds(r0, K)])
        return carry

    lax.fori_loop(0, NCHUNK, chunk, 0)


_sc_fin = pl.kernel(
    _sc_fin_body,
    out_type=jax.ShapeDtypeStruct((E, 128), jnp.float32),
    mesh=_MESH,
    compiler_params=_SC_PARAMS,
    scratch_types=[
        pltpu.VMEM((K,), jnp.int32),
        pltpu.VMEM((K,), jnp.int32),
        pltpu.VMEM((K, 128), jnp.float32),
        pltpu.VMEM((K, 128), jnp.float32),
        pltpu.VMEM((K, 128), jnp.float32),
        pltpu.SemaphoreType.DMA,
    ],
)


_TC1_R = 2000


def _tc1_body(x_ref, wl_ref, bl_ref, wr_ref, br_ref, att_ref,
              xl_ref, xr_ref, selfr_ref):
    x = x_ref[...]
    xl = jnp.dot(x, wl_ref[...], preferred_element_type=jnp.float32,
                 precision=lax.Precision.HIGHEST) + bl_ref[...]
    xr = jnp.dot(x, wr_ref[...], preferred_element_type=jnp.float32,
                 precision=lax.Precision.HIGHEST) + br_ref[...]
    xl_ref[...] = xl
    xr_ref[...] = xr
    z = xl + xr
    z = jnp.maximum(z, 0.0) + 0.2 * jnp.minimum(z, 0.0)
    g = jnp.exp(jnp.dot(z, att_ref[...], preferred_element_type=jnp.float32,
                 precision=lax.Precision.HIGHEST))
    selfr_ref[...] = jnp.concatenate(
        [g * xl, g, jnp.zeros((_TC1_R, TW - C1 - 1), jnp.float32)], axis=1)


def _tc2_body(pa_ref, selfr_ref, h1aug_ref):
    t = (pa_ref[0] + pa_ref[1])[:N] + selfr_ref[...]
    num = t[:, :C1]
    den = t[:, C1:C1 + 1]
    gat = num / den
    gat = gat - jnp.mean(gat, axis=0, keepdims=True)
    h1 = jnp.maximum(
        gat / jnp.sqrt(1e-5 + jnp.mean(jnp.sum(gat * gat, axis=-1))), 0.0)
    rows = jnp.concatenate(
        [h1, jnp.ones((N, 1), jnp.float32),
         jnp.zeros((N, TW - C1 - 1), jnp.float32)], axis=1)
    h1aug_ref[...] = jnp.concatenate(
        [rows, jnp.zeros((NPAD - N, TW), jnp.float32)], axis=0)


def _tc3_body(s1_ref, s2_ref, h1aug_ref, w1_ref, b1_ref, fc_ref,
              t1_ref, t2_ref):
    s1 = s1_ref[0] + s1_ref[1]
    s2 = s2_ref[0] + s2_ref[1]
    h1 = h1aug_ref[:, :C1]
    deg_s = s1[:, C1:C1 + 1]
    deg_d = s2[:, C1:C1 + 1]
    agg = jnp.concatenate([deg_s * h1 + s2[:, :C1],
                           s1[:, :C1] + deg_d * h1], axis=1)
    w1 = w1_ref[...]
    wa, wb, wc = w1[:64], w1[64:128], w1[128:192]
    wx = wa - wb - wc
    fc = fc_ref[...]
    cs = (jnp.dot(h1, wx[:C1], preferred_element_type=jnp.float32,
                 precision=lax.Precision.HIGHEST)
          + jnp.dot(agg, wb, preferred_element_type=jnp.float32,
                 precision=lax.Precision.HIGHEST) + b1_ref[...])
    cd = (jnp.dot(h1, wx[C1:], preferred_element_type=jnp.float32,
                 precision=lax.Precision.HIGHEST)
          + jnp.dot(agg, wc, preferred_element_type=jnp.float32,
                 precision=lax.Precision.HIGHEST))
    ds = jnp.dot(h1, fc[:C1], preferred_element_type=jnp.float32,
                 precision=lax.Precision.HIGHEST)
    dd = jnp.dot(h1, fc[C1:], preferred_element_type=jnp.float32,
                 precision=lax.Precision.HIGHEST)
    t1_ref[...] = jnp.concatenate([cs, ds], axis=1)
    t2_ref[...] = jnp.concatenate([cd, dd], axis=1)


_TC3_R = 1280
_TC4_R = 4000


def _tc4_body(uw_ref, w2_ref, fc_ref, fcb_ref, b2_ref, ow_ref, ob_ref,
              out_ref):
    fc = fc_ref[...]
    m = jnp.dot(w2_ref[...], fc, preferred_element_type=jnp.float32,
                 precision=lax.Precision.HIGHEST)
    c0 = jnp.dot(b2_ref[...], fc, preferred_element_type=jnp.float32,
                 precision=lax.Precision.HIGHEST) + fcb_ref[...]
    uw = uw_ref[...]
    u = uw[:, :64]
    w = uw[:, 64:]
    v = jnp.maximum(jnp.dot(u, m, preferred_element_type=jnp.float32,
                 precision=lax.Precision.HIGHEST) + w + c0,
                    0.0)
    out_ref[...] = (jnp.dot(v, ow_ref[...], preferred_element_type=jnp.float32,
                 precision=lax.Precision.HIGHEST)
                    + ob_ref[...])


def _full_spec(shape):
    return pl.BlockSpec(shape, lambda i: tuple(0 for _ in shape))


@jax.jit
def kernel(x, edge_index, edge_attr, Wl1, bl1, Wr1, br1, att1, bias1,
           mp_W1, mp_b1, mp_W2, mp_b2, fc_W1, fc_b1, out_W, out_b):
    del edge_attr, bias1
    src = edge_index[0]
    dst = edge_index[1]
    zer = jnp.zeros((NPAD, TW), jnp.float32)

    xl, xr, selfr = pl.pallas_call(
        _tc1_body,
        grid=(N // _TC1_R,),
        in_specs=[
            pl.BlockSpec((_TC1_R, 128), lambda i: (i, 0)),
            _full_spec((128, C1)),
            _full_spec((1, C1)),
            _full_spec((128, C1)),
            _full_spec((1, C1)),
            _full_spec((C1, 1)),
        ],
        out_specs=[pl.BlockSpec((_TC1_R, C1), lambda i: (i, 0)),
                   pl.BlockSpec((_TC1_R, C1), lambda i: (i, 0)),
                   pl.BlockSpec((_TC1_R, TW), lambda i: (i, 0))],
        out_shape=[jax.ShapeDtypeStruct((N, C1), jnp.float32),
                   jax.ShapeDtypeStruct((N, C1), jnp.float32),
                   jax.ShapeDtypeStruct((N, TW), jnp.float32)],
    )(x, Wl1, bl1.reshape(1, C1), Wr1, br1.reshape(1, C1),
      att1.reshape(C1, 1))

    pa = _sc_gat(src, dst, xl, xr, att1, zer)

    h1aug = pl.pallas_call(
        _tc2_body,
        out_shape=jax.ShapeDtypeStruct((NPAD, TW), jnp.float32),
    )(pa, selfr)

    s1, s2 = _sc_agg(src, dst, h1aug, zer)

    t1, t2 = pl.pallas_call(
        _tc3_body,
        grid=(NPAD // _TC3_R,),
        in_specs=[
            pl.BlockSpec((NC, _TC3_R, TW), lambda i: (0, i, 0)),
            pl.BlockSpec((NC, _TC3_R, TW), lambda i: (0, i, 0)),
            pl.BlockSpec((_TC3_R, TW), lambda i: (i, 0)),
            _full_spec((192, 64)),
            _full_spec((1, 64)),
            _full_spec((64, 64)),
        ],
        out_specs=[pl.BlockSpec((_TC3_R, 128), lambda i: (i, 0)),
                   pl.BlockSpec((_TC3_R, 128), lambda i: (i, 0))],
        out_shape=[jax.ShapeDtypeStruct((NPAD, 128), jnp.float32),
                   jax.ShapeDtypeStruct((NPAD, 128), jnp.float32)],
    )(s1, s2, h1aug, mp_W1, mp_b1.reshape(1, 64), fc_W1)

    uw = _sc_fin(src, dst, t1, t2)

    out = pl.pallas_call(
        _tc4_body,
        grid=(E // _TC4_R,),
        in_specs=[
            pl.BlockSpec((_TC4_R, 128), lambda i: (i, 0)),
            _full_spec((64, 64)),
            _full_spec((64, 64)),
            _full_spec((1, 64)),
            _full_spec((1, 64)),
            _full_spec((64, 16)),
            _full_spec((1, 16)),
        ],
        out_specs=pl.BlockSpec((_TC4_R, 16), lambda i: (i, 0)),
        out_shape=jax.ShapeDtypeStruct((E, 16), jnp.float32),
    )(uw, mp_W2, fc_W1, fc_b1.reshape(1, 64), mp_b2.reshape(1, 64),
      out_W, out_b.reshape(1, 16))

    return out


# speedup vs baseline: 8.3298x; 1.5488x over previous
import functools

import jax
import jax.numpy as jnp
from jax import lax
from jax.experimental import pallas as pl
from jax.experimental.pallas import tpu as pltpu
from jax.experimental.pallas import tpu_sc as plsc

N = 10000
NPAD = 10240
E = 320000
C1 = 32
TW = 48
NC, NS, L = 2, 16, 16
NW = NC * NS
EPW = E // NW
K = 40
NCHUNK = EPW // K
RING = 5
NGRP = NCHUNK // RING
RPT = NPAD // NS

_MESH = plsc.VectorSubcoreMesh(core_axis_name="c", subcore_axis_name="s",
                               num_cores=NC, num_subcores=NS)
_SC_PARAMS = pltpu.CompilerParams(needs_layout_passes=False,
                                  use_tc_tiling_on_sc=False)


def _sc_gat_body(idx_hbm, xl_hbm, xr_hbm, att_hbm, zer_hbm, out_hbm,
                 idxv, gbs, gbd, srow, attv, table, semg, sems):
    cid = lax.axis_index("c")
    sid = lax.axis_index("s")
    wid = sid * NC + cid
    pltpu.sync_copy(zer_hbm.at[pl.ds(sid * RPT, RPT)],
                    table.at[pl.ds(sid * RPT, RPT)])
    pltpu.sync_copy(att_hbm, attv)
    pltpu.sync_copy(idx_hbm.at[wid], idxv)
    plsc.subcore_barrier()

    zero16 = jnp.zeros((L,), jnp.float32)
    oneh = jnp.where(lax.iota(jnp.int32, L) == 0, 1.0, 0.0)

    def grp(g, carry):
        for b in range(RING):
            c = g * RING + b

            @pl.when(g > 0)
            def _():
                pltpu.make_async_copy(srow.at[b], table.at[idxv.at[0, 1]],
                                      sems.at[b]).wait()

            pltpu.async_copy(xl_hbm.at[idxv.at[c, 0]], gbs.at[b], semg.at[b])
            pltpu.async_copy(xr_hbm.at[idxv.at[c, 1]], gbd.at[b], semg.at[b])
        for b in range(RING):
            c = g * RING + b
            pltpu.make_async_copy(xl_hbm.at[idxv.at[0, 0]], gbs.at[b],
                                  semg.at[b]).wait()
            pltpu.make_async_copy(xr_hbm.at[idxv.at[0, 1]], gbd.at[b],
                                  semg.at[b]).wait()
            a0 = attv[pl.ds(0, L)]
            a1 = attv[pl.ds(L, L)]

            def edge(i, carry2):
                s0 = gbs[b, i, pl.ds(0, L)]
                s1 = gbs[b, i, pl.ds(L, L)]
                z0 = s0 + gbd[b, i, pl.ds(0, L)]
                z1 = s1 + gbd[b, i, pl.ds(L, L)]
                z0 = jnp.maximum(z0, 0.0) + 0.2 * jnp.minimum(z0, 0.0)
                z1 = jnp.maximum(z1, 0.0) + 0.2 * jnp.minimum(z1, 0.0)
                tot = jnp.sum(z0 * a0 + z1 * a1)
                gv = jnp.exp(tot + zero16)
                srow[b, i, pl.ds(0, L)] = gv * s0
                srow[b, i, pl.ds(L, L)] = gv * s1
                srow[b, i, pl.ds(2 * L, L)] = gv * oneh
                return carry2

            lax.fori_loop(0, K, edge, 0)
            pltpu.async_copy(srow.at[b], table.at[idxv.at[c, 1]], sems.at[b],
                             add=True)
        return carry

    lax.fori_loop(0, NGRP, grp, 0)
    for b in range(RING):
        pltpu.make_async_copy(srow.at[b], table.at[idxv.at[0, 1]],
                              sems.at[b]).wait()
    plsc.subcore_barrier()
    pltpu.sync_copy(table.at[pl.ds(sid * RPT, RPT)],
                    out_hbm.at[cid, pl.ds(sid * RPT, RPT)])


_sc_gat = pl.kernel(
    _sc_gat_body,
    out_type=jax.ShapeDtypeStruct((NC, NPAD, TW), jnp.float32),
    mesh=_MESH,
    compiler_params=_SC_PARAMS,
    scratch_types=[
        pltpu.VMEM((NCHUNK, 2, K), jnp.int32),
        pltpu.VMEM((RING, K, C1), jnp.float32),
        pltpu.VMEM((RING, K, C1), jnp.float32),
        pltpu.VMEM((RING, K, TW), jnp.float32),
        pltpu.VMEM((C1,), jnp.float32),
        pltpu.VMEM_SHARED((NPAD, TW), jnp.float32),
        pltpu.SemaphoreType.DMA((RING,)),
        pltpu.SemaphoreType.DMA((RING,)),
    ],
)


def _sc_agg_body(idx_hbm, h1_hbm, zer_hbm, o1_hbm, o2_hbm,
                 idxv, bufs, bufd, t1, t2, semg, sems):
    cid = lax.axis_index("c")
    sid = lax.axis_index("s")
    wid = sid * NC + cid
    pltpu.sync_copy(zer_hbm.at[pl.ds(sid * RPT, RPT)],
                    t1.at[pl.ds(sid * RPT, RPT)])
    pltpu.sync_copy(zer_hbm.at[pl.ds(sid * RPT, RPT)],
                    t2.at[pl.ds(sid * RPT, RPT)])
    pltpu.sync_copy(idx_hbm.at[wid], idxv)
    plsc.subcore_barrier()

    def grp(g, carry):
        for b in range(RING):
            c = g * RING + b

            @pl.when(g > 0)
            def _():
                pltpu.make_async_copy(bufd.at[b], t1.at[idxv.at[0, 0]],
                                      sems.at[b]).wait()
                pltpu.make_async_copy(bufs.at[b], t2.at[idxv.at[0, 1]],
                                      sems.at[b]).wait()

            pltpu.async_copy(h1_hbm.at[idxv.at[c, 1]], bufd.at[b], semg.at[b])
            pltpu.async_copy(h1_hbm.at[idxv.at[c, 0]], bufs.at[b], semg.at[b])
        for b in range(RING):
            c = g * RING + b
            pltpu.make_async_copy(h1_hbm.at[idxv.at[0, 1]], bufd.at[b],
                                  semg.at[b]).wait()
            pltpu.make_async_copy(h1_hbm.at[idxv.at[0, 0]], bufs.at[b],
                                  semg.at[b]).wait()
            pltpu.async_copy(bufd.at[b], t1.at[idxv.at[c, 0]], sems.at[b],
                             add=True)
            pltpu.async_copy(bufs.at[b], t2.at[idxv.at[c, 1]], sems.at[b],
                             add=True)
        return carry

    lax.fori_loop(0, NGRP, grp, 0)
    for b in range(RING):
        pltpu.make_async_copy(bufd.at[b], t1.at[idxv.at[0, 0]],
                              sems.at[b]).wait()
        pltpu.make_async_copy(bufs.at[b], t2.at[idxv.at[0, 1]],
                              sems.at[b]).wait()
    plsc.subcore_barrier()
    pltpu.sync_copy(t1.at[pl.ds(sid * RPT, RPT)],
                    o1_hbm.at[cid, pl.ds(sid * RPT, RPT)])
    pltpu.sync_copy(t2.at[pl.ds(sid * RPT, RPT)],
                    o2_hbm.at[cid, pl.ds(sid * RPT, RPT)])


_sc_agg = pl.kernel(
    _sc_agg_body,
    out_type=[jax.ShapeDtypeStruct((NC, NPAD, TW), jnp.float32),
              jax.ShapeDtypeStruct((NC, NPAD, TW), jnp.float32)],
    mesh=_MESH,
    compiler_params=_SC_PARAMS,
    scratch_types=[
        pltpu.VMEM((NCHUNK, 2, K), jnp.int32),
        pltpu.VMEM((RING, K, TW), jnp.float32),
        pltpu.VMEM((RING, K, TW), jnp.float32),
        pltpu.VMEM_SHARED((NPAD, TW), jnp.float32),
        pltpu.VMEM_SHARED((NPAD, TW), jnp.float32),
        pltpu.SemaphoreType.DMA((RING,)),
        pltpu.SemaphoreType.DMA((RING,)),
    ],
)


def _sc_fin_body(idx_hbm, t1_hbm, t2_hbm, out_hbm,
                 idxv, bufa, bufb, obuf, semg, semw):
    cid = lax.axis_index("c")
    sid = lax.axis_index("s")
    wid = sid * NC + cid
    base = wid * EPW
    pltpu.sync_copy(idx_hbm.at[wid], idxv)

    def grp(g, carry):
        for b in range(RING):
            c = g * RING + b

            @pl.when(g > 0)
            def _():
                pltpu.make_async_copy(obuf.at[b], out_hbm.at[pl.ds(0, K)],
                                      semw.at[b]).wait()

            pltpu.async_copy(t1_hbm.at[idxv.at[c, 0]], bufa.at[b], semg.at[b])
            pltpu.async_copy(t2_hbm.at[idxv.at[c, 1]], bufb.at[b], semg.at[b])
        for b in range(RING):
            c = g * RING + b
            pltpu.make_async_copy(t1_hbm.at[idxv.at[0, 0]], bufa.at[b],
                                  semg.at[b]).wait()
            pltpu.make_async_copy(t2_hbm.at[idxv.at[0, 1]], bufb.at[b],
                                  semg.at[b]).wait()

            def edge(i, carry2):
                for j in range(4):
                    v = (bufa[b, i, pl.ds(j * L, L)]
                         + bufb[b, i, pl.ds(j * L, L)])
                    obuf[b, i, pl.ds(j * L, L)] = jnp.maximum(v, 0.0)
                for j in range(4, 8):
                    obuf[b, i, pl.ds(j * L, L)] = (
                        bufa[b, i, pl.ds(j * L, L)]
                        + bufb[b, i, pl.ds(j * L, L)])
                return carry2

            lax.fori_loop(0, K, edge, 0)
            pltpu.async_copy(obuf.at[b], out_hbm.at[pl.ds(base + c * K, K)],
                             semw.at[b])
        return carry

    lax.fori_loop(0, NGRP, grp, 0)
    for b in range(RING):
        pltpu.make_async_copy(obuf.at[b], out_hbm.at[pl.ds(0, K)],
                              semw.at[b]).wait()


_sc_fin = pl.kernel(
    _sc_fin_body,
    out_type=jax.ShapeDtypeStruct((E, 128), jnp.float32),
    mesh=_MESH,
    compiler_params=_SC_PARAMS,
    scratch_types=[
        pltpu.VMEM((NCHUNK, 2, K), jnp.int32),
        pltpu.VMEM((RING, K, 128), jnp.float32),
        pltpu.VMEM((RING, K, 128), jnp.float32),
        pltpu.VMEM((RING, K, 128), jnp.float32),
        pltpu.SemaphoreType.DMA((RING,)),
        pltpu.SemaphoreType.DMA((RING,)),
    ],
)


_TC1_R = 2000


def _tc1_body(x_ref, wl_ref, bl_ref, wr_ref, br_ref, att_ref,
              xl_ref, xr_ref, selfr_ref):
    x = x_ref[...]
    xl = jnp.dot(x, wl_ref[...], preferred_element_type=jnp.float32,
                 precision=lax.Precision.HIGHEST) + bl_ref[...]
    xr = jnp.dot(x, wr_ref[...], preferred_element_type=jnp.float32,
                 precision=lax.Precision.HIGHEST) + br_ref[...]
    xl_ref[...] = xl
    xr_ref[...] = xr
    z = xl + xr
    z = jnp.maximum(z, 0.0) + 0.2 * jnp.minimum(z, 0.0)
    g = jnp.exp(jnp.dot(z, att_ref[...], preferred_element_type=jnp.float32,
                        precision=lax.Precision.HIGHEST))
    selfr_ref[...] = jnp.concatenate(
        [g * xl, g, jnp.zeros((_TC1_R, TW - C1 - 1), jnp.float32)], axis=1)


def _tc2_body(pa_ref, selfr_ref, h1aug_ref):
    t = (pa_ref[0] + pa_ref[1])[:N] + selfr_ref[...]
    num = t[:, :C1]
    den = t[:, C1:C1 + 1]
    gat = num / den
    gat = gat - jnp.mean(gat, axis=0, keepdims=True)
    h1 = jnp.maximum(
        gat / jnp.sqrt(1e-5 + jnp.mean(jnp.sum(gat * gat, axis=-1))), 0.0)
    rows = jnp.concatenate(
        [h1, jnp.ones((N, 1), jnp.float32),
         jnp.zeros((N, TW - C1 - 1), jnp.float32)], axis=1)
    h1aug_ref[...] = jnp.concatenate(
        [rows, jnp.zeros((NPAD - N, TW), jnp.float32)], axis=0)


def _tc3_body(s1_ref, s2_ref, h1aug_ref, w1_ref, b1_ref, fc_ref,
              t1_ref, t2_ref):
    s1 = s1_ref[0] + s1_ref[1]
    s2 = s2_ref[0] + s2_ref[1]
    h1 = h1aug_ref[:, :C1]
    deg_s = s1[:, C1:C1 + 1]
    deg_d = s2[:, C1:C1 + 1]
    agg = jnp.concatenate([deg_s * h1 + s2[:, :C1],
                           s1[:, :C1] + deg_d * h1], axis=1)
    w1 = w1_ref[...]
    wa, wb, wc = w1[:64], w1[64:128], w1[128:192]
    wx = wa - wb - wc
    fc = fc_ref[...]
    cs = (jnp.dot(h1, wx[:C1], preferred_element_type=jnp.float32,
                  precision=lax.Precision.HIGHEST)
          + jnp.dot(agg, wb, preferred_element_type=jnp.float32,
                    precision=lax.Precision.HIGHEST) + b1_ref[...])
    cd = (jnp.dot(h1, wx[C1:], preferred_element_type=jnp.float32,
                  precision=lax.Precision.HIGHEST)
          + jnp.dot(agg, wc, preferred_element_type=jnp.float32,
                    precision=lax.Precision.HIGHEST))
    ds = jnp.dot(h1, fc[:C1], preferred_element_type=jnp.float32,
                 precision=lax.Precision.HIGHEST)
    dd = jnp.dot(h1, fc[C1:], preferred_element_type=jnp.float32,
                 precision=lax.Precision.HIGHEST)
    t1_ref[...] = jnp.concatenate([cs, ds], axis=1)
    t2_ref[...] = jnp.concatenate([cd, dd], axis=1)


_TC3_R = 1280
_TC4_R = 4000


def _tc4_body(uw_ref, w2_ref, fc_ref, fcb_ref, b2_ref, ow_ref, ob_ref,
              out_ref):
    fc = fc_ref[...]
    m = jnp.dot(w2_ref[...], fc, preferred_element_type=jnp.float32,
                precision=lax.Precision.HIGHEST)
    c0 = jnp.dot(b2_ref[...], fc, preferred_element_type=jnp.float32,
                 precision=lax.Precision.HIGHEST) + fcb_ref[...]
    uw = uw_ref[...]
    u = uw[:, :64]
    w = uw[:, 64:]
    v = jnp.maximum(jnp.dot(u, m, preferred_element_type=jnp.float32,
                            precision=lax.Precision.HIGHEST) + w + c0, 0.0)
    out_ref[...] = (jnp.dot(v, ow_ref[...], preferred_element_type=jnp.float32,
                            precision=lax.Precision.HIGHEST) + ob_ref[...])


def _full_spec(shape):
    return pl.BlockSpec(shape, lambda i: tuple(0 for _ in shape))


@jax.jit
def kernel(x, edge_index, edge_attr, Wl1, bl1, Wr1, br1, att1, bias1,
           mp_W1, mp_b1, mp_W2, mp_b2, fc_W1, fc_b1, out_W, out_b):
    del edge_attr, bias1
    src = edge_index[0]
    dst = edge_index[1]
    idx3 = jnp.stack([src.reshape(NW, NCHUNK, K),
                      dst.reshape(NW, NCHUNK, K)], axis=2)
    zer = jnp.zeros((NPAD, TW), jnp.float32)

    xl, xr, selfr = pl.pallas_call(
        _tc1_body,
        grid=(N // _TC1_R,),
        in_specs=[
            pl.BlockSpec((_TC1_R, 128), lambda i: (i, 0)),
            _full_spec((128, C1)),
            _full_spec((1, C1)),
            _full_spec((128, C1)),
            _full_spec((1, C1)),
            _full_spec((C1, 1)),
        ],
        out_specs=[pl.BlockSpec((_TC1_R, C1), lambda i: (i, 0)),
                   pl.BlockSpec((_TC1_R, C1), lambda i: (i, 0)),
                   pl.BlockSpec((_TC1_R, TW), lambda i: (i, 0))],
        out_shape=[jax.ShapeDtypeStruct((N, C1), jnp.float32),
                   jax.ShapeDtypeStruct((N, C1), jnp.float32),
                   jax.ShapeDtypeStruct((N, TW), jnp.float32)],
    )(x, Wl1, bl1.reshape(1, C1), Wr1, br1.reshape(1, C1),
      att1.reshape(C1, 1))

    pa = _sc_gat(idx3, xl, xr, att1, zer)

    h1aug = pl.pallas_call(
        _tc2_body,
        out_shape=jax.ShapeDtypeStruct((NPAD, TW), jnp.float32),
    )(pa, selfr)

    s1, s2 = _sc_agg(idx3, h1aug, zer)

    t1, t2 = pl.pallas_call(
        _tc3_body,
        grid=(NPAD // _TC3_R,),
        in_specs=[
            pl.BlockSpec((NC, _TC3_R, TW), lambda i: (0, i, 0)),
            pl.BlockSpec((NC, _TC3_R, TW), lambda i: (0, i, 0)),
            pl.BlockSpec((_TC3_R, TW), lambda i: (i, 0)),
            _full_spec((192, 64)),
            _full_spec((1, 64)),
            _full_spec((64, 64)),
        ],
        out_specs=[pl.BlockSpec((_TC3_R, 128), lambda i: (i, 0)),
                   pl.BlockSpec((_TC3_R, 128), lambda i: (i, 0))],
        out_shape=[jax.ShapeDtypeStruct((NPAD, 128), jnp.float32),
                   jax.ShapeDtypeStruct((NPAD, 128), jnp.float32)],
    )(s1, s2, h1aug, mp_W1, mp_b1.reshape(1, 64), fc_W1)

    uw = _sc_fin(idx3, t1, t2)

    out = pl.pallas_call(
        _tc4_body,
        grid=(E // _TC4_R,),
        in_specs=[
            pl.BlockSpec((_TC4_R, 128), lambda i: (i, 0)),
            _full_spec((64, 64)),
            _full_spec((64, 64)),
            _full_spec((1, 64)),
            _full_spec((1, 64)),
            _full_spec((64, 16)),
            _full_spec((1, 16)),
        ],
        out_specs=pl.BlockSpec((_TC4_R, 16), lambda i: (i, 0)),
        out_shape=jax.ShapeDtypeStruct((E, 16), jnp.float32),
    )(uw, mp_W2, fc_W1, fc_b1.reshape(1, 64), mp_b2.reshape(1, 64),
      out_W, out_b.reshape(1, 16))

    return out
